# baseline (speedup 1.0000x reference)
import functools

import jax
import jax.numpy as jnp
from jax.experimental import pallas as pl

NUM_Q = 4
K = 1024
D = 256
BETA = 0.25
BN = 512


def _rvq_kernel(x_ref, cbm2t_ref, cbn_ref, c1_ref, c2_ref, c3_ref,
                xq_ref, loss_ref, idx_ref, *, n_total):
    i = pl.program_id(0)

    r = x_ref[...]
    rn = jnp.sum(r * r, axis=1, keepdims=True)
    xq_acc = jnp.zeros_like(r)
    loss_acc = jnp.zeros((), jnp.float32)
    idx_cols = []
    for s in range(NUM_Q):
        sm2 = jnp.dot(r, cbm2t_ref[s],
                      preferred_element_type=jnp.float32)
        d = (rn + cbn_ref[s]) + sm2
        m = jnp.min(d, axis=1, keepdims=True)
        iota = jax.lax.broadcasted_iota(jnp.int32, d.shape, 1)
        idx = jnp.min(jnp.where(d == m, iota, K), axis=1, keepdims=True)
        onehot = (iota == idx).astype(jnp.bfloat16)
        dot = lambda a, b: jnp.dot(a, b, preferred_element_type=jnp.float32)
        xq = (dot(onehot, c1_ref[s]) + dot(onehot, c2_ref[s])) \
            + dot(onehot, c3_ref[s])
        t = xq - r
        x_res = r + t
        loss_acc = loss_acc + jnp.sum(t * t)
        r = r - x_res
        rn = jnp.sum(r * r, axis=1, keepdims=True)
        xq_acc = xq_acc + x_res
        idx_cols.append(idx)

    xq_ref[...] = xq_acc
    idx_ref[...] = jnp.concatenate(idx_cols, axis=1)

    scale = (1.0 + BETA) / (NUM_Q * n_total * D)

    @pl.when(i == 0)
    def _():
        loss_ref[...] = jnp.zeros((1, 1), jnp.float32)

    loss_ref[...] += (loss_acc * scale)[None, None]


def kernel(x, codebooks):
    n = x.shape[0]
    nb = n // BN

    cbm2t = jnp.transpose(-2.0 * codebooks, (0, 2, 1))
    cbn = jnp.sum(codebooks * codebooks, axis=2)[:, None, :]
    barrier = jax.lax.optimization_barrier
    c1 = codebooks.astype(jnp.bfloat16)
    rem = barrier(codebooks - barrier(c1).astype(jnp.float32))
    c2 = rem.astype(jnp.bfloat16)
    c3 = barrier(rem - barrier(c2).astype(jnp.float32)).astype(jnp.bfloat16)

    xq, loss, idx = pl.pallas_call(
        functools.partial(_rvq_kernel, n_total=n),
        grid=(nb,),
        in_specs=[
            pl.BlockSpec((BN, D), lambda i: (i, 0)),
            pl.BlockSpec((NUM_Q, D, K), lambda i: (0, 0, 0)),
            pl.BlockSpec((NUM_Q, 1, K), lambda i: (0, 0, 0)),
            pl.BlockSpec((NUM_Q, K, D), lambda i: (0, 0, 0)),
            pl.BlockSpec((NUM_Q, K, D), lambda i: (0, 0, 0)),
            pl.BlockSpec((NUM_Q, K, D), lambda i: (0, 0, 0)),
        ],
        out_specs=[
            pl.BlockSpec((BN, D), lambda i: (i, 0)),
            pl.BlockSpec((1, 1), lambda i: (0, 0)),
            pl.BlockSpec((BN, NUM_Q), lambda i: (i, 0)),
        ],
        out_shape=[
            jax.ShapeDtypeStruct((n, D), jnp.float32),
            jax.ShapeDtypeStruct((1, 1), jnp.float32),
            jax.ShapeDtypeStruct((n, NUM_Q), jnp.int32),
        ],
    )(x, cbm2t, cbn, c1, c2, c3)
    return xq, loss[0, 0], idx


# speedup vs baseline: 1.6328x; 1.6328x over previous
import functools

import jax
import jax.numpy as jnp
from jax.experimental import pallas as pl
from jax.experimental.pallas import tpu as pltpu

NUM_Q = 4
K = 1024
D = 256
BETA = 0.25
BN = 512


def _rvq_kernel(x_ref, cb_ref, xq_ref, loss_ref, idx_ref,
                cbm2t_scr, cbn_scr, c1_scr, c2_scr, c3_scr, *, n_total):
    i = pl.program_id(0)

    @pl.when(i == 0)
    def _prep():
        for s in range(NUM_Q):
            cb = cb_ref[s]
            cbm2t_scr[s] = -2.0 * cb.T
            cbn_scr[s] = jnp.sum(cb * cb, axis=1, keepdims=True).T
            c1 = cb.astype(jnp.bfloat16)
            rem = cb - c1.astype(jnp.float32)
            c2 = rem.astype(jnp.bfloat16)
            c3 = (rem - c2.astype(jnp.float32)).astype(jnp.bfloat16)
            c1_scr[s] = c1
            c2_scr[s] = c2
            c3_scr[s] = c3

    H = BN // 2
    rs = [x_ref[:H, :], x_ref[H:, :]]
    rns = [jnp.sum(r * r, axis=1, keepdims=True) for r in rs]
    xq_accs = [jnp.zeros_like(r) for r in rs]
    loss_acc = jnp.zeros((), jnp.float32)
    idx_cols = [[], []]
    for s in range(NUM_Q):
        for h in range(2):
            r = rs[h]
            sm2 = jnp.dot(r, cbm2t_scr[s],
                          preferred_element_type=jnp.float32)
            d = (rns[h] + cbn_scr[s]) + sm2
            idx = jax.lax.argmin(d, axis=1, index_dtype=jnp.int32)[:, None]
            iota = jax.lax.broadcasted_iota(jnp.int32, d.shape, 1)
            onehot = (iota == idx).astype(jnp.bfloat16)
            dot = lambda a, b: jnp.dot(a, b,
                                       preferred_element_type=jnp.float32)
            xq = (dot(onehot, c1_scr[s]) + dot(onehot, c2_scr[s])) \
                + dot(onehot, c3_scr[s])
            t = xq - r
            x_res = r + t
            loss_acc = loss_acc + jnp.sum(t * t)
            rs[h] = r - x_res
            rns[h] = jnp.sum(rs[h] * rs[h], axis=1, keepdims=True)
            xq_accs[h] = xq_accs[h] + x_res
            idx_cols[h].append(idx)

    xq_ref[:H, :] = xq_accs[0]
    xq_ref[H:, :] = xq_accs[1]
    idx_ref[:H, :] = jnp.concatenate(idx_cols[0], axis=1)
    idx_ref[H:, :] = jnp.concatenate(idx_cols[1], axis=1)

    scale = (1.0 + BETA) / (NUM_Q * n_total * D)

    @pl.when(i == 0)
    def _():
        loss_ref[...] = jnp.zeros((1, 1), jnp.float32)

    loss_ref[...] += (loss_acc * scale)[None, None]


def kernel(x, codebooks):
    n = x.shape[0]
    nb = n // BN

    xq, loss, idx = pl.pallas_call(
        functools.partial(_rvq_kernel, n_total=n),
        grid=(nb,),
        in_specs=[
            pl.BlockSpec((BN, D), lambda i: (i, 0)),
            pl.BlockSpec((NUM_Q, K, D), lambda i: (0, 0, 0)),
        ],
        out_specs=[
            pl.BlockSpec((BN, D), lambda i: (i, 0)),
            pl.BlockSpec((1, 1), lambda i: (0, 0)),
            pl.BlockSpec((BN, NUM_Q), lambda i: (i, 0)),
        ],
        out_shape=[
            jax.ShapeDtypeStruct((n, D), jnp.float32),
            jax.ShapeDtypeStruct((1, 1), jnp.float32),
            jax.ShapeDtypeStruct((n, NUM_Q), jnp.int32),
        ],
        scratch_shapes=[
            pltpu.VMEM((NUM_Q, D, K), jnp.float32),
            pltpu.VMEM((NUM_Q, 1, K), jnp.float32),
            pltpu.VMEM((NUM_Q, K, D), jnp.bfloat16),
            pltpu.VMEM((NUM_Q, K, D), jnp.bfloat16),
            pltpu.VMEM((NUM_Q, K, D), jnp.bfloat16),
        ],
    )(x, codebooks)
    return xq, loss[0, 0], idx


# speedup vs baseline: 1.6381x; 1.0033x over previous
import functools

import jax
import jax.numpy as jnp
from jax.experimental import pallas as pl
from jax.experimental.pallas import tpu as pltpu

NUM_Q = 4
K = 1024
D = 256
BETA = 0.25
BN = 512


def _rvq_kernel(x_ref, cb_ref, xq_ref, loss_ref, idx_ref,
                cbm2t_scr, cbn_scr, csplit_scr, *, n_total):
    i = pl.program_id(0)

    @pl.when(i == 0)
    def _prep():
        for s in range(NUM_Q):
            cb = cb_ref[s]
            cbm2t_scr[s] = -2.0 * cb.T
            cbn_scr[s] = jnp.sum(cb * cb, axis=1, keepdims=True).T
            c1 = cb.astype(jnp.bfloat16)
            rem = cb - c1.astype(jnp.float32)
            c2 = rem.astype(jnp.bfloat16)
            c3 = (rem - c2.astype(jnp.float32)).astype(jnp.bfloat16)
            csplit_scr[s, :, :D] = c1
            csplit_scr[s, :, D:2 * D] = c2
            csplit_scr[s, :, 2 * D:] = c3

    NCH = 2
    H = BN // NCH
    rs = [x_ref[c * H:(c + 1) * H, :] for c in range(NCH)]
    rns = [jnp.sum(r * r, axis=1, keepdims=True) for r in rs]
    xq_accs = [jnp.zeros_like(r) for r in rs]
    loss_acc = jnp.zeros((), jnp.float32)
    idx_cols = [[] for _ in range(NCH)]
    for s in range(NUM_Q):
        for h in range(NCH):
            r = rs[h]
            sm2 = jnp.dot(r, cbm2t_scr[s],
                          preferred_element_type=jnp.float32)
            d = (rns[h] + cbn_scr[s]) + sm2
            idx = jax.lax.argmin(d, axis=1, index_dtype=jnp.int32)[:, None]
            iota = jax.lax.broadcasted_iota(jnp.int32, d.shape, 1)
            onehot = (iota == idx).astype(jnp.bfloat16)
            xq3 = jnp.dot(onehot, csplit_scr[s],
                          preferred_element_type=jnp.float32)
            xq = (xq3[:, :D] + xq3[:, D:2 * D]) + xq3[:, 2 * D:]
            t = xq - r
            x_res = r + t
            loss_acc = loss_acc + jnp.sum(t * t)
            rs[h] = r - x_res
            rns[h] = jnp.sum(rs[h] * rs[h], axis=1, keepdims=True)
            xq_accs[h] = xq_accs[h] + x_res
            idx_cols[h].append(idx)

    for c in range(NCH):
        xq_ref[c * H:(c + 1) * H, :] = xq_accs[c]
        idx_ref[c * H:(c + 1) * H, :] = jnp.concatenate(idx_cols[c], axis=1)

    scale = (1.0 + BETA) / (NUM_Q * n_total * D)

    @pl.when(i == 0)
    def _():
        loss_ref[...] = jnp.zeros((1, 1), jnp.float32)

    loss_ref[...] += (loss_acc * scale)[None, None]


def kernel(x, codebooks):
    n = x.shape[0]
    nb = n // BN

    xq, loss, idx = pl.pallas_call(
        functools.partial(_rvq_kernel, n_total=n),
        grid=(nb,),
        in_specs=[
            pl.BlockSpec((BN, D), lambda i: (i, 0)),
            pl.BlockSpec((NUM_Q, K, D), lambda i: (0, 0, 0)),
        ],
        out_specs=[
            pl.BlockSpec((BN, D), lambda i: (i, 0)),
            pl.BlockSpec((1, 1), lambda i: (0, 0)),
            pl.BlockSpec((BN, NUM_Q), lambda i: (i, 0)),
        ],
        out_shape=[
            jax.ShapeDtypeStruct((n, D), jnp.float32),
            jax.ShapeDtypeStruct((1, 1), jnp.float32),
            jax.ShapeDtypeStruct((n, NUM_Q), jnp.int32),
        ],
        scratch_shapes=[
            pltpu.VMEM((NUM_Q, D, K), jnp.float32),
            pltpu.VMEM((NUM_Q, 1, K), jnp.float32),
            pltpu.VMEM((NUM_Q, K, 3 * D), jnp.bfloat16),
        ],
    )(x, codebooks)
    return xq, loss[0, 0], idx
